# baseline (speedup 1.0000x reference)
import functools

import jax
import jax.numpy as jnp
from jax import lax
from jax.experimental import pallas as pl
from jax.experimental.pallas import tpu as pltpu
from jax.experimental.pallas import tpu_sc as plsc

N_NODES = 10000
N_EDGES = 320000
D = 128
DH = 64
NC = 2
NT = 16
EPT = N_EDGES // NT
CHUNK = 80
NCHUNK = EPT // CHUNK
SB = 25
NSB = NCHUNK // SB
ROWS_PT = 624
TAIL = N_NODES - NT * ROWS_PT
ZR = 104
RB = 1000
_F32 = jnp.float32
_I32 = jnp.int32


def _sc_body(xtab, src_a, src_b, dst2, p2,
             out_sum, out_p, out_deg,
             acc_sum, acc_p, acc_deg,
             srcv, dstv, pv, rows2, scaled2, zbuf, zd, ones,
             gsem, ssem, psem, dsem):
    c = lax.axis_index("c")
    s = lax.axis_index("s")

    for q in range(CHUNK // 16):
        ones[pl.ds(q * 16, 16)] = jnp.ones((16,), _F32)

    def _zb(i, carry):
        for q in range(DH // 16):
            zbuf[i, pl.ds(q * 16, 16)] = jnp.zeros((16,), _F32)
        return carry
    lax.fori_loop(0, ZR, _zb, 0)

    def _zd(i, carry):
        zd[pl.ds(i * 16, 16)] = jnp.zeros((16,), _F32)
        return carry
    lax.fori_loop(0, 1280 // 16, _zd, 0)

    for q in range(ROWS_PT // ZR):
        pltpu.sync_copy(zbuf, acc_sum.at[pl.ds(s * ROWS_PT + q * ZR, ZR)])
        pltpu.sync_copy(zbuf, acc_p.at[pl.ds(s * ROWS_PT + q * ZR, ZR)])

    @pl.when(s == 0)
    def _():
        pltpu.sync_copy(zbuf.at[pl.ds(0, TAIL)],
                        acc_sum.at[pl.ds(NT * ROWS_PT, TAIL)])
        pltpu.sync_copy(zbuf.at[pl.ds(0, TAIL)],
                        acc_p.at[pl.ds(NT * ROWS_PT, TAIL)])

    @pl.when(s < 7)
    def _():
        pltpu.sync_copy(zd, acc_deg.at[pl.ds(s * 1280, 1280)])

    @pl.when(s == 7)
    def _():
        pltpu.sync_copy(zd.at[pl.ds(0, 1040)], acc_deg.at[pl.ds(7 * 1280, 1040)])

    plsc.subcore_barrier()

    def _wait_scatters(par, j):
        pltpu.make_async_copy(rows2.at[par], acc_sum.at[dstv.at[j]],
                              ssem.at[par]).wait()
        pltpu.make_async_copy(scaled2.at[par], acc_p.at[dstv.at[j]],
                              psem.at[par]).wait()
        pltpu.make_async_copy(ones, acc_deg.at[dstv.at[j]],
                              dsem.at[par]).wait()

    def super_body(b, carry):
        @pl.when(c == 0)
        def _():
            pltpu.sync_copy(src_a.at[s, pl.ds(b * SB, SB)], srcv)

        @pl.when(c == 1)
        def _():
            pltpu.sync_copy(src_b.at[s, pl.ds(b * SB, SB)], srcv)

        pltpu.sync_copy(dst2.at[s, pl.ds(b * SB, SB)], dstv)
        pltpu.sync_copy(p2.at[s, pl.ds(b * SB, SB)], pv)

        pltpu.async_copy(xtab.at[srcv.at[0]], rows2.at[0], gsem.at[0])

        def chunk_body(j, carry2):
            par = lax.rem(j, 2)
            npar = 1 - par

            @pl.when(j > 0)
            def _():
                _wait_scatters(npar, j - 1)

            @pl.when(j + 1 < SB)
            def _():
                pltpu.async_copy(xtab.at[srcv.at[j + 1]], rows2.at[npar],
                                 gsem.at[npar])

            pltpu.make_async_copy(xtab.at[srcv.at[j]], rows2.at[par],
                                  gsem.at[par]).wait()

            for g in range(CHUNK // 16):
                pvec = pv[j, pl.ds(g * 16, 16)]
                for i in range(16):
                    e = g * 16 + i
                    pb = jnp.full((16,), pvec[i], _F32)
                    for q in range(DH // 16):
                        scaled2[par, e, pl.ds(q * 16, 16)] = (
                            rows2[par, e, pl.ds(q * 16, 16)] * pb)

            pltpu.async_copy(rows2.at[par], acc_sum.at[dstv.at[j]],
                             ssem.at[par], add=True)
            pltpu.async_copy(scaled2.at[par], acc_p.at[dstv.at[j]],
                             psem.at[par], add=True)
            pltpu.async_copy(ones, acc_deg.at[dstv.at[j]],
                             dsem.at[par], add=True)
            return carry2

        lax.fori_loop(0, SB, chunk_body, 0)
        _wait_scatters((SB - 1) % 2, SB - 1)
        return carry

    lax.fori_loop(0, NSB, super_body, 0)

    plsc.subcore_barrier()

    r0 = s * ROWS_PT
    pltpu.sync_copy(acc_sum.at[pl.ds(r0, ROWS_PT)],
                    out_sum.at[c, pl.ds(r0, ROWS_PT)])
    pltpu.sync_copy(acc_p.at[pl.ds(r0, ROWS_PT)],
                    out_p.at[c, pl.ds(r0, ROWS_PT)])

    @pl.when(s == 0)
    def _():
        pltpu.sync_copy(acc_sum.at[pl.ds(NT * ROWS_PT, TAIL)],
                        out_sum.at[c, pl.ds(NT * ROWS_PT, TAIL)])
        pltpu.sync_copy(acc_p.at[pl.ds(NT * ROWS_PT, TAIL)],
                        out_p.at[c, pl.ds(NT * ROWS_PT, TAIL)])

    @pl.when(jnp.logical_and(c == 0, s < 7))
    def _():
        pltpu.sync_copy(acc_deg.at[pl.ds(s * 1280, 1280)],
                        out_deg.at[pl.ds(s * 1280, 1280)])

    @pl.when(jnp.logical_and(c == 0, s == 7))
    def _():
        pltpu.sync_copy(acc_deg.at[pl.ds(7 * 1280, 1040)],
                        out_deg.at[pl.ds(7 * 1280, 1040)])


def _sc_scatter(xtab, src_a, src_b, dst2, p2):
    mesh = plsc.VectorSubcoreMesh(core_axis_name="c", subcore_axis_name="s")
    f = pl.kernel(
        _sc_body,
        mesh=mesh,
        compiler_params=pltpu.CompilerParams(needs_layout_passes=False,
                                             use_tc_tiling_on_sc=False),
        out_type=[
            jax.ShapeDtypeStruct((NC, N_NODES, DH), _F32),
            jax.ShapeDtypeStruct((NC, N_NODES, DH), _F32),
            jax.ShapeDtypeStruct((N_NODES,), _F32),
        ],
        scratch_types=[
            pltpu.VMEM_SHARED((N_NODES, DH), _F32),
            pltpu.VMEM_SHARED((N_NODES, DH), _F32),
            pltpu.VMEM_SHARED((N_NODES,), _F32),
            pltpu.VMEM((SB, CHUNK), _I32),
            pltpu.VMEM((SB, CHUNK), _I32),
            pltpu.VMEM((SB, CHUNK), _F32),
            pltpu.VMEM((2, CHUNK, DH), _F32),
            pltpu.VMEM((2, CHUNK, DH), _F32),
            pltpu.VMEM((ZR, DH), _F32),
            pltpu.VMEM((1280,), _F32),
            pltpu.VMEM((CHUNK,), _F32),
            pltpu.SemaphoreType.DMA((2,)),
            pltpu.SemaphoreType.DMA((2,)),
            pltpu.SemaphoreType.DMA((2,)),
            pltpu.SemaphoreType.DMA((2,)),
        ],
    )
    return f(xtab, src_a, src_b, dst2, p2)


def _tc_body(s2, pa2, dga, xr, wcat, wroot, brow, out):
    cat = jnp.concatenate([s2[0], s2[1], pa2[0], pa2[1]], axis=1)
    deg = dga[...]
    agg = jnp.dot(cat, wcat[...], preferred_element_type=_F32)
    agg = agg / jnp.maximum(deg, 1.0)
    out[...] = (agg
                + jnp.dot(xr[...], wroot[...], preferred_element_type=_F32)
                + brow[...])


def _tc_combine(sum2, p2, dega, x, wcat, wroot, brow):
    grid = (N_NODES // RB,)
    return pl.pallas_call(
        _tc_body,
        grid=grid,
        in_specs=[
            pl.BlockSpec((NC, RB, DH), lambda i: (0, i, 0)),
            pl.BlockSpec((NC, RB, DH), lambda i: (0, i, 0)),
            pl.BlockSpec((RB, 1), lambda i: (i, 0)),
            pl.BlockSpec((RB, D), lambda i: (i, 0)),
            pl.BlockSpec((2 * D, D), lambda i: (0, 0)),
            pl.BlockSpec((D, D), lambda i: (0, 0)),
            pl.BlockSpec((1, D), lambda i: (0, 0)),
        ],
        out_specs=pl.BlockSpec((RB, D), lambda i: (i, 0)),
        out_shape=jax.ShapeDtypeStruct((N_NODES, D), _F32),
    )(sum2, p2, dega, x, wcat, wroot, brow)


def kernel(x, edge_index, edge_attr, weight, root, bias):
    src = edge_index[0]
    dst = edge_index[1]
    p = edge_attr[:, 0]
    xtab = jnp.concatenate([x[:, :DH], x[:, DH:]], axis=0)
    src_a = src.reshape(NT, NCHUNK, CHUNK)
    src_b = (src + N_NODES).reshape(NT, NCHUNK, CHUNK)
    dst2 = dst.reshape(NT, NCHUNK, CHUNK)
    p2 = p.reshape(NT, NCHUNK, CHUNK)

    sum2, pacc2, deg = _sc_scatter(xtab, src_a, src_b, dst2, p2)

    wcat = jnp.concatenate([weight[0], weight[1] - weight[0]], axis=0)
    return _tc_combine(sum2, pacc2, deg.reshape(N_NODES, 1),
                       x, wcat, root, bias.reshape(1, D))


# speedup vs baseline: 1.0454x; 1.0454x over previous
import functools

import jax
import jax.numpy as jnp
from jax import lax
from jax.experimental import pallas as pl
from jax.experimental.pallas import tpu as pltpu
from jax.experimental.pallas import tpu_sc as plsc

N_NODES = 10000
N_EDGES = 320000
D = 128
DH = 64
NC = 2
NT = 16
EPT = N_EDGES // NT
CHUNK = 80
NCHUNK = EPT // CHUNK
SB = 25
NSB = NCHUNK // SB
ROWS_PT = 624
TAIL = N_NODES - NT * ROWS_PT
ZR = 104
RB = 1000
_F32 = jnp.float32
_I32 = jnp.int32


def _sc_body(xtab, ei4, p2,
             out_sum, out_p, out_deg_a, out_deg_b,
             acc_sum, acc_p, acc_deg,
             srcv, dstv, pv, rows2, scaled2, zbuf, zd, ones,
             gsem, ssem, psem, dsem):
    c = lax.axis_index("c")
    s = lax.axis_index("s")

    for q in range(CHUNK // 16):
        ones[pl.ds(q * 16, 16)] = jnp.ones((16,), _F32)

    def _zb(i, carry):
        for q in range(DH // 16):
            zbuf[i, pl.ds(q * 16, 16)] = jnp.zeros((16,), _F32)
        return carry
    lax.fori_loop(0, ZR, _zb, 0)

    def _zd(i, carry):
        zd[pl.ds(i * 16, 16)] = jnp.zeros((16,), _F32)
        return carry
    lax.fori_loop(0, 1280 // 16, _zd, 0)

    for q in range(ROWS_PT // ZR):
        pltpu.sync_copy(zbuf, acc_sum.at[pl.ds(s * ROWS_PT + q * ZR, ZR)])
        pltpu.sync_copy(zbuf, acc_p.at[pl.ds(s * ROWS_PT + q * ZR, ZR)])

    @pl.when(s == 0)
    def _():
        pltpu.sync_copy(zbuf.at[pl.ds(0, TAIL)],
                        acc_sum.at[pl.ds(NT * ROWS_PT, TAIL)])
        pltpu.sync_copy(zbuf.at[pl.ds(0, TAIL)],
                        acc_p.at[pl.ds(NT * ROWS_PT, TAIL)])

    @pl.when(s < 7)
    def _():
        pltpu.sync_copy(zd, acc_deg.at[pl.ds(s * 1280, 1280)])

    @pl.when(s == 7)
    def _():
        pltpu.sync_copy(zd.at[pl.ds(0, 1040)], acc_deg.at[pl.ds(7 * 1280, 1040)])

    plsc.subcore_barrier()

    def _wait_scatters(par, j, b):
        pltpu.make_async_copy(rows2.at[par], acc_sum.at[dstv.at[j]],
                              ssem.at[par]).wait()
        pltpu.make_async_copy(scaled2.at[par], acc_p.at[dstv.at[j]],
                              psem.at[par]).wait()

        @pl.when(lax.rem(b * SB + j, 2) == c)
        def _():
            pltpu.make_async_copy(ones, acc_deg.at[dstv.at[j]],
                                  dsem.at[par]).wait()

    nsplat = jnp.full((16,), N_NODES, _I32)

    def super_body(b, carry):
        pltpu.sync_copy(ei4.at[0, s, pl.ds(b * SB, SB)], srcv)
        pltpu.sync_copy(ei4.at[1, s, pl.ds(b * SB, SB)], dstv)
        pltpu.sync_copy(p2.at[s, pl.ds(b * SB, SB)], pv)

        @pl.when(c == 1)
        def _():
            def _adj(r, carry2):
                for q in range(CHUNK // 16):
                    srcv[r, pl.ds(q * 16, 16)] = (
                        srcv[r, pl.ds(q * 16, 16)] + nsplat)
                return carry2
            lax.fori_loop(0, SB, _adj, 0)

        pltpu.async_copy(xtab.at[srcv.at[0]], rows2.at[0], gsem.at[0])

        def chunk_body(j, carry2):
            par = lax.rem(j, 2)
            npar = 1 - par

            @pl.when(j > 0)
            def _():
                _wait_scatters(npar, j - 1, b)

            @pl.when(j + 1 < SB)
            def _():
                pltpu.async_copy(xtab.at[srcv.at[j + 1]], rows2.at[npar],
                                 gsem.at[npar])

            pltpu.make_async_copy(xtab.at[srcv.at[j]], rows2.at[par],
                                  gsem.at[par]).wait()

            for g in range(CHUNK // 16):
                pvec = pv[j, pl.ds(g * 16, 16)]
                for i in range(16):
                    e = g * 16 + i
                    pb = jnp.full((16,), pvec[i], _F32)
                    for q in range(DH // 16):
                        scaled2[par, e, pl.ds(q * 16, 16)] = (
                            rows2[par, e, pl.ds(q * 16, 16)] * pb)

            pltpu.async_copy(rows2.at[par], acc_sum.at[dstv.at[j]],
                             ssem.at[par], add=True)
            pltpu.async_copy(scaled2.at[par], acc_p.at[dstv.at[j]],
                             psem.at[par], add=True)

            @pl.when(lax.rem(b * SB + j, 2) == c)
            def _():
                pltpu.async_copy(ones, acc_deg.at[dstv.at[j]],
                                 dsem.at[par], add=True)

            return carry2

        lax.fori_loop(0, SB, chunk_body, 0)
        _wait_scatters((SB - 1) % 2, SB - 1, b)
        return carry

    lax.fori_loop(0, NSB, super_body, 0)

    plsc.subcore_barrier()

    r0 = s * ROWS_PT
    pltpu.sync_copy(acc_sum.at[pl.ds(r0, ROWS_PT)],
                    out_sum.at[c, pl.ds(r0, ROWS_PT)])
    pltpu.sync_copy(acc_p.at[pl.ds(r0, ROWS_PT)],
                    out_p.at[c, pl.ds(r0, ROWS_PT)])

    @pl.when(s == 0)
    def _():
        pltpu.sync_copy(acc_sum.at[pl.ds(NT * ROWS_PT, TAIL)],
                        out_sum.at[c, pl.ds(NT * ROWS_PT, TAIL)])
        pltpu.sync_copy(acc_p.at[pl.ds(NT * ROWS_PT, TAIL)],
                        out_p.at[c, pl.ds(NT * ROWS_PT, TAIL)])

    @pl.when(jnp.logical_and(c == 0, s < 7))
    def _():
        pltpu.sync_copy(acc_deg.at[pl.ds(s * 1280, 1280)],
                        out_deg_a.at[pl.ds(s * 1280, 1280)])

    @pl.when(jnp.logical_and(c == 0, s == 7))
    def _():
        pltpu.sync_copy(acc_deg.at[pl.ds(7 * 1280, 1040)],
                        out_deg_a.at[pl.ds(7 * 1280, 1040)])

    @pl.when(jnp.logical_and(c == 1, s < 7))
    def _():
        pltpu.sync_copy(acc_deg.at[pl.ds(s * 1280, 1280)],
                        out_deg_b.at[pl.ds(s * 1280, 1280)])

    @pl.when(jnp.logical_and(c == 1, s == 7))
    def _():
        pltpu.sync_copy(acc_deg.at[pl.ds(7 * 1280, 1040)],
                        out_deg_b.at[pl.ds(7 * 1280, 1040)])


def _sc_scatter(xtab, ei4, p2):
    mesh = plsc.VectorSubcoreMesh(core_axis_name="c", subcore_axis_name="s")
    f = pl.kernel(
        _sc_body,
        mesh=mesh,
        compiler_params=pltpu.CompilerParams(needs_layout_passes=False,
                                             use_tc_tiling_on_sc=False),
        out_type=[
            jax.ShapeDtypeStruct((NC, N_NODES, DH), _F32),
            jax.ShapeDtypeStruct((NC, N_NODES, DH), _F32),
            jax.ShapeDtypeStruct((N_NODES,), _F32),
            jax.ShapeDtypeStruct((N_NODES,), _F32),
        ],
        scratch_types=[
            pltpu.VMEM_SHARED((N_NODES, DH), _F32),
            pltpu.VMEM_SHARED((N_NODES, DH), _F32),
            pltpu.VMEM_SHARED((N_NODES,), _F32),
            pltpu.VMEM((SB, CHUNK), _I32),
            pltpu.VMEM((SB, CHUNK), _I32),
            pltpu.VMEM((SB, CHUNK), _F32),
            pltpu.VMEM((2, CHUNK, DH), _F32),
            pltpu.VMEM((2, CHUNK, DH), _F32),
            pltpu.VMEM((ZR, DH), _F32),
            pltpu.VMEM((1280,), _F32),
            pltpu.VMEM((CHUNK,), _F32),
            pltpu.SemaphoreType.DMA((2,)),
            pltpu.SemaphoreType.DMA((2,)),
            pltpu.SemaphoreType.DMA((2,)),
            pltpu.SemaphoreType.DMA((2,)),
        ],
    )
    return f(xtab, ei4, p2)


def _tc_body(s2, pa2, dga, dgb, xr, wcat, wroot, brow, out):
    cat = jnp.concatenate([s2[0], s2[1], pa2[0], pa2[1]], axis=1)
    deg = dga[...] + dgb[...]
    agg = jnp.dot(cat, wcat[...], preferred_element_type=_F32)
    agg = agg / jnp.maximum(deg, 1.0)
    out[...] = (agg
                + jnp.dot(xr[...], wroot[...], preferred_element_type=_F32)
                + brow[...])


def _tc_combine(sum2, p2, dega, degb, x, wcat, wroot, brow):
    grid = (N_NODES // RB,)
    return pl.pallas_call(
        _tc_body,
        grid=grid,
        in_specs=[
            pl.BlockSpec((NC, RB, DH), lambda i: (0, i, 0)),
            pl.BlockSpec((NC, RB, DH), lambda i: (0, i, 0)),
            pl.BlockSpec((RB, 1), lambda i: (i, 0)),
            pl.BlockSpec((RB, 1), lambda i: (i, 0)),
            pl.BlockSpec((RB, D), lambda i: (i, 0)),
            pl.BlockSpec((2 * D, D), lambda i: (0, 0)),
            pl.BlockSpec((D, D), lambda i: (0, 0)),
            pl.BlockSpec((1, D), lambda i: (0, 0)),
        ],
        out_specs=pl.BlockSpec((RB, D), lambda i: (i, 0)),
        out_shape=jax.ShapeDtypeStruct((N_NODES, D), _F32),
    )(sum2, p2, dega, degb, x, wcat, wroot, brow)


def kernel(x, edge_index, edge_attr, weight, root, bias):
    p = edge_attr[:, 0]
    xtab = jnp.concatenate([x[:, :DH], x[:, DH:]], axis=0)
    ei4 = edge_index.reshape(2, NT, NCHUNK, CHUNK)
    p2 = p.reshape(NT, NCHUNK, CHUNK)

    sum2, pacc2, deg_a, deg_b = _sc_scatter(xtab, ei4, p2)

    wcat = jnp.concatenate([weight[0], weight[1] - weight[0]], axis=0)
    return _tc_combine(sum2, pacc2,
                       deg_a.reshape(N_NODES, 1), deg_b.reshape(N_NODES, 1),
                       x, wcat, root, bias.reshape(1, D))


# speedup vs baseline: 1.0920x; 1.0446x over previous
import functools

import jax
import jax.numpy as jnp
from jax import lax
from jax.experimental import pallas as pl
from jax.experimental.pallas import tpu as pltpu
from jax.experimental.pallas import tpu_sc as plsc

N_NODES = 10000
N_EDGES = 320000
D = 128
DH = 64
NC = 2
NT = 16
EPT = N_EDGES // NT
CHUNK = 80
NCHUNK = EPT // CHUNK
SB = 25
NSB = NCHUNK // SB
ROWS_PT = 624
TAIL = N_NODES - NT * ROWS_PT
ZR = 104
RB = 1000
_F32 = jnp.float32
_I32 = jnp.int32


def _sc_body(xtab, ei4, p2,
             out_sum, out_p, out_deg_a, out_deg_b,
             acc_sum, acc_p, acc_deg,
             srcv, dstv, pv, rows2, scaled2, zbuf, zd, ones,
             gsem, ssem, psem, dsem, stsem):
    c = lax.axis_index("c")
    s = lax.axis_index("s")

    for q in range(CHUNK // 16):
        ones[pl.ds(q * 16, 16)] = jnp.ones((16,), _F32)

    def _zb(i, carry):
        for q in range(DH // 16):
            zbuf[i, pl.ds(q * 16, 16)] = jnp.zeros((16,), _F32)
        return carry
    lax.fori_loop(0, ZR, _zb, 0)

    def _zd(i, carry):
        zd[pl.ds(i * 16, 16)] = jnp.zeros((16,), _F32)
        return carry
    lax.fori_loop(0, 1280 // 16, _zd, 0)

    for q in range(ROWS_PT // ZR):
        pltpu.sync_copy(zbuf, acc_sum.at[pl.ds(s * ROWS_PT + q * ZR, ZR)])
        pltpu.sync_copy(zbuf, acc_p.at[pl.ds(s * ROWS_PT + q * ZR, ZR)])

    @pl.when(s == 0)
    def _():
        pltpu.sync_copy(zbuf.at[pl.ds(0, TAIL)],
                        acc_sum.at[pl.ds(NT * ROWS_PT, TAIL)])
        pltpu.sync_copy(zbuf.at[pl.ds(0, TAIL)],
                        acc_p.at[pl.ds(NT * ROWS_PT, TAIL)])

    @pl.when(s < 7)
    def _():
        pltpu.sync_copy(zd, acc_deg.at[pl.ds(s * 1280, 1280)])

    @pl.when(s == 7)
    def _():
        pltpu.sync_copy(zd.at[pl.ds(0, 1040)], acc_deg.at[pl.ds(7 * 1280, 1040)])

    plsc.subcore_barrier()

    def _wait_scatters(sp, par, j, b):
        pltpu.make_async_copy(rows2.at[par], acc_sum.at[dstv.at[sp, j]],
                              ssem.at[par]).wait()
        pltpu.make_async_copy(scaled2.at[par], acc_p.at[dstv.at[sp, j]],
                              psem.at[par]).wait()

        @pl.when(lax.rem(b * SB + j, 2) == c)
        def _():
            pltpu.make_async_copy(ones, acc_deg.at[dstv.at[sp, j]],
                                  dsem.at[par]).wait()

    nsplat = jnp.full((16,), N_NODES, _I32)

    def _stage(b, sp):
        pltpu.async_copy(ei4.at[0, s, pl.ds(b * SB, SB)], srcv.at[sp],
                         stsem.at[sp])
        pltpu.async_copy(ei4.at[1, s, pl.ds(b * SB, SB)], dstv.at[sp],
                         stsem.at[sp])
        pltpu.async_copy(p2.at[s, pl.ds(b * SB, SB)], pv.at[sp],
                         stsem.at[sp])

    def _wait_stage(b, sp):
        pltpu.make_async_copy(ei4.at[0, s, pl.ds(b * SB, SB)], srcv.at[sp],
                              stsem.at[sp]).wait()
        pltpu.make_async_copy(ei4.at[1, s, pl.ds(b * SB, SB)], dstv.at[sp],
                              stsem.at[sp]).wait()
        pltpu.make_async_copy(p2.at[s, pl.ds(b * SB, SB)], pv.at[sp],
                              stsem.at[sp]).wait()

    _stage(0, 0)

    def super_body(b, carry):
        sp = lax.rem(b, 2)
        _wait_stage(b, sp)

        @pl.when(b + 1 < NSB)
        def _():
            _stage(b + 1, 1 - sp)

        @pl.when(c == 1)
        def _():
            def _adj(r, carry2):
                for q in range(CHUNK // 16):
                    srcv[sp, r, pl.ds(q * 16, 16)] = (
                        srcv[sp, r, pl.ds(q * 16, 16)] + nsplat)
                return carry2
            lax.fori_loop(0, SB, _adj, 0)

        pltpu.async_copy(xtab.at[srcv.at[sp, 0]], rows2.at[0], gsem.at[0])

        def chunk_body(j, carry2):
            par = lax.rem(j, 2)
            npar = 1 - par

            @pl.when(j > 0)
            def _():
                _wait_scatters(sp, npar, j - 1, b)

            @pl.when(j + 1 < SB)
            def _():
                pltpu.async_copy(xtab.at[srcv.at[sp, j + 1]], rows2.at[npar],
                                 gsem.at[npar])

            pltpu.make_async_copy(xtab.at[srcv.at[sp, j]], rows2.at[par],
                                  gsem.at[par]).wait()

            for g in range(CHUNK // 16):
                pvec = pv[sp, j, pl.ds(g * 16, 16)]
                for i in range(16):
                    e = g * 16 + i
                    pb = jnp.full((16,), pvec[i], _F32)
                    for q in range(DH // 16):
                        scaled2[par, e, pl.ds(q * 16, 16)] = (
                            rows2[par, e, pl.ds(q * 16, 16)] * pb)

            pltpu.async_copy(rows2.at[par], acc_sum.at[dstv.at[sp, j]],
                             ssem.at[par], add=True)
            pltpu.async_copy(scaled2.at[par], acc_p.at[dstv.at[sp, j]],
                             psem.at[par], add=True)

            @pl.when(lax.rem(b * SB + j, 2) == c)
            def _():
                pltpu.async_copy(ones, acc_deg.at[dstv.at[sp, j]],
                                 dsem.at[par], add=True)

            return carry2

        lax.fori_loop(0, SB, chunk_body, 0)
        _wait_scatters(sp, (SB - 1) % 2, SB - 1, b)
        return carry

    lax.fori_loop(0, NSB, super_body, 0)

    plsc.subcore_barrier()

    r0 = s * ROWS_PT
    pltpu.sync_copy(acc_sum.at[pl.ds(r0, ROWS_PT)],
                    out_sum.at[c, pl.ds(r0, ROWS_PT)])
    pltpu.sync_copy(acc_p.at[pl.ds(r0, ROWS_PT)],
                    out_p.at[c, pl.ds(r0, ROWS_PT)])

    @pl.when(s == 0)
    def _():
        pltpu.sync_copy(acc_sum.at[pl.ds(NT * ROWS_PT, TAIL)],
                        out_sum.at[c, pl.ds(NT * ROWS_PT, TAIL)])
        pltpu.sync_copy(acc_p.at[pl.ds(NT * ROWS_PT, TAIL)],
                        out_p.at[c, pl.ds(NT * ROWS_PT, TAIL)])

    @pl.when(jnp.logical_and(c == 0, s < 7))
    def _():
        pltpu.sync_copy(acc_deg.at[pl.ds(s * 1280, 1280)],
                        out_deg_a.at[pl.ds(s * 1280, 1280)])

    @pl.when(jnp.logical_and(c == 0, s == 7))
    def _():
        pltpu.sync_copy(acc_deg.at[pl.ds(7 * 1280, 1040)],
                        out_deg_a.at[pl.ds(7 * 1280, 1040)])

    @pl.when(jnp.logical_and(c == 1, s < 7))
    def _():
        pltpu.sync_copy(acc_deg.at[pl.ds(s * 1280, 1280)],
                        out_deg_b.at[pl.ds(s * 1280, 1280)])

    @pl.when(jnp.logical_and(c == 1, s == 7))
    def _():
        pltpu.sync_copy(acc_deg.at[pl.ds(7 * 1280, 1040)],
                        out_deg_b.at[pl.ds(7 * 1280, 1040)])


def _sc_scatter(xtab, ei4, p2):
    mesh = plsc.VectorSubcoreMesh(core_axis_name="c", subcore_axis_name="s")
    f = pl.kernel(
        _sc_body,
        mesh=mesh,
        compiler_params=pltpu.CompilerParams(needs_layout_passes=False,
                                             use_tc_tiling_on_sc=False),
        out_type=[
            jax.ShapeDtypeStruct((NC, N_NODES, DH), _F32),
            jax.ShapeDtypeStruct((NC, N_NODES, DH), _F32),
            jax.ShapeDtypeStruct((N_NODES,), _F32),
            jax.ShapeDtypeStruct((N_NODES,), _F32),
        ],
        scratch_types=[
            pltpu.VMEM_SHARED((N_NODES, DH), _F32),
            pltpu.VMEM_SHARED((N_NODES, DH), _F32),
            pltpu.VMEM_SHARED((N_NODES,), _F32),
            pltpu.VMEM((2, SB, CHUNK), _I32),
            pltpu.VMEM((2, SB, CHUNK), _I32),
            pltpu.VMEM((2, SB, CHUNK), _F32),
            pltpu.VMEM((2, CHUNK, DH), _F32),
            pltpu.VMEM((2, CHUNK, DH), _F32),
            pltpu.VMEM((ZR, DH), _F32),
            pltpu.VMEM((1280,), _F32),
            pltpu.VMEM((CHUNK,), _F32),
            pltpu.SemaphoreType.DMA((2,)),
            pltpu.SemaphoreType.DMA((2,)),
            pltpu.SemaphoreType.DMA((2,)),
            pltpu.SemaphoreType.DMA((2,)),
            pltpu.SemaphoreType.DMA((2,)),
        ],
    )
    return f(xtab, ei4, p2)


def _tc_body(s2, pa2, dga, dgb, xr, wcat, wroot, brow, out):
    cat = jnp.concatenate([s2[0], s2[1], pa2[0], pa2[1]], axis=1)
    deg = dga[...] + dgb[...]
    agg = jnp.dot(cat, wcat[...], preferred_element_type=_F32)
    agg = agg / jnp.maximum(deg, 1.0)
    out[...] = (agg
                + jnp.dot(xr[...], wroot[...], preferred_element_type=_F32)
                + brow[...])


def _tc_combine(sum2, p2, dega, degb, x, wcat, wroot, brow):
    grid = (N_NODES // RB,)
    return pl.pallas_call(
        _tc_body,
        grid=grid,
        in_specs=[
            pl.BlockSpec((NC, RB, DH), lambda i: (0, i, 0)),
            pl.BlockSpec((NC, RB, DH), lambda i: (0, i, 0)),
            pl.BlockSpec((RB, 1), lambda i: (i, 0)),
            pl.BlockSpec((RB, 1), lambda i: (i, 0)),
            pl.BlockSpec((RB, D), lambda i: (i, 0)),
            pl.BlockSpec((2 * D, D), lambda i: (0, 0)),
            pl.BlockSpec((D, D), lambda i: (0, 0)),
            pl.BlockSpec((1, D), lambda i: (0, 0)),
        ],
        out_specs=pl.BlockSpec((RB, D), lambda i: (i, 0)),
        out_shape=jax.ShapeDtypeStruct((N_NODES, D), _F32),
    )(sum2, p2, dega, degb, x, wcat, wroot, brow)


def kernel(x, edge_index, edge_attr, weight, root, bias):
    p = edge_attr[:, 0]
    xtab = jnp.concatenate([x[:, :DH], x[:, DH:]], axis=0)
    ei4 = edge_index.reshape(2, NT, NCHUNK, CHUNK)
    p2 = p.reshape(NT, NCHUNK, CHUNK)

    sum2, pacc2, deg_a, deg_b = _sc_scatter(xtab, ei4, p2)

    wcat = jnp.concatenate([weight[0], weight[1] - weight[0]], axis=0)
    return _tc_combine(sum2, pacc2,
                       deg_a.reshape(N_NODES, 1), deg_b.reshape(N_NODES, 1),
                       x, wcat, root, bias.reshape(1, D))


# speedup vs baseline: 1.1381x; 1.0423x over previous
import functools

import jax
import jax.numpy as jnp
from jax import lax
from jax.experimental import pallas as pl
from jax.experimental.pallas import tpu as pltpu
from jax.experimental.pallas import tpu_sc as plsc

N_NODES = 10000
N_EDGES = 320000
D = 128
DH = 64
NC = 2
NT = 16
CHUNK = 128
NCHUNK = 160
EPTP = NCHUNK * CHUNK
E_PAD = NT * EPTP
PAD_ROWS = 32
SB = 16
NSB = NCHUNK // SB
ROWS_PT = 624
TAIL = N_NODES - NT * ROWS_PT
ZR = 48
RB = 1000
_F32 = jnp.float32
_I32 = jnp.int32


def _sc_body(xtab, ei4, p2,
             out_sum, out_p, out_deg_a, out_deg_b,
             acc_sum, acc_p, acc_deg,
             srcv, dstv, pv, rows2, scaled2, zbuf, zd, ones,
             gsem, ssem, psem, dsem, stsem):
    c = lax.axis_index("c")
    s = lax.axis_index("s")

    for q in range(CHUNK // 16):
        ones[pl.ds(q * 16, 16)] = jnp.ones((16,), _F32)

    def _zb(i, carry):
        for q in range(DH // 16):
            zbuf[i, pl.ds(q * 16, 16)] = jnp.zeros((16,), _F32)
        return carry
    lax.fori_loop(0, ZR, _zb, 0)

    def _zd(i, carry):
        zd[pl.ds(i * 16, 16)] = jnp.zeros((16,), _F32)
        return carry
    lax.fori_loop(0, 1280 // 16, _zd, 0)

    for q in range(ROWS_PT // ZR):
        pltpu.sync_copy(zbuf, acc_sum.at[pl.ds(s * ROWS_PT + q * ZR, ZR)])
        pltpu.sync_copy(zbuf, acc_p.at[pl.ds(s * ROWS_PT + q * ZR, ZR)])

    @pl.when(s == 0)
    def _():
        pltpu.sync_copy(zbuf.at[pl.ds(0, TAIL)],
                        acc_sum.at[pl.ds(NT * ROWS_PT, TAIL)])
        pltpu.sync_copy(zbuf.at[pl.ds(0, TAIL)],
                        acc_p.at[pl.ds(NT * ROWS_PT, TAIL)])

    @pl.when(s < 7)
    def _():
        pltpu.sync_copy(zd, acc_deg.at[pl.ds(s * 1280, 1280)])

    @pl.when(s == 7)
    def _():
        pltpu.sync_copy(zd.at[pl.ds(0, 1040)], acc_deg.at[pl.ds(7 * 1280, 1040)])

    plsc.subcore_barrier()

    def _wait_scatters(sp, par, j, b):
        pltpu.make_async_copy(rows2.at[par], acc_sum.at[dstv.at[sp, j]],
                              ssem.at[par]).wait()
        pltpu.make_async_copy(scaled2.at[par], acc_p.at[dstv.at[sp, j]],
                              psem.at[par]).wait()

        @pl.when(lax.rem(b * SB + j, 2) == c)
        def _():
            pltpu.make_async_copy(ones, acc_deg.at[dstv.at[sp, j]],
                                  dsem.at[par]).wait()

    nsplat = jnp.full((16,), N_NODES, _I32)

    def _stage(b, sp):
        pltpu.async_copy(ei4.at[0, s, pl.ds(b * SB, SB)], srcv.at[sp],
                         stsem.at[sp])
        pltpu.async_copy(ei4.at[1, s, pl.ds(b * SB, SB)], dstv.at[sp],
                         stsem.at[sp])
        pltpu.async_copy(p2.at[s, pl.ds(b * SB, SB)], pv.at[sp],
                         stsem.at[sp])

    def _wait_stage(b, sp):
        pltpu.make_async_copy(ei4.at[0, s, pl.ds(b * SB, SB)], srcv.at[sp],
                              stsem.at[sp]).wait()
        pltpu.make_async_copy(ei4.at[1, s, pl.ds(b * SB, SB)], dstv.at[sp],
                              stsem.at[sp]).wait()
        pltpu.make_async_copy(p2.at[s, pl.ds(b * SB, SB)], pv.at[sp],
                              stsem.at[sp]).wait()

    _stage(0, 0)

    def super_body(b, carry):
        sp = lax.rem(b, 2)
        _wait_stage(b, sp)

        @pl.when(b + 1 < NSB)
        def _():
            _stage(b + 1, 1 - sp)

        @pl.when(c == 1)
        def _():
            def _adj(r, carry2):
                for q in range(CHUNK // 16):
                    srcv[sp, r, pl.ds(q * 16, 16)] = (
                        srcv[sp, r, pl.ds(q * 16, 16)] + nsplat)
                return carry2
            lax.fori_loop(0, SB, _adj, 0)

        pltpu.async_copy(xtab.at[srcv.at[sp, 0]], rows2.at[0], gsem.at[0])

        def chunk_body(j, carry2):
            par = lax.rem(j, 2)
            npar = 1 - par

            @pl.when(j > 0)
            def _():
                _wait_scatters(sp, npar, j - 1, b)

            @pl.when(j + 1 < SB)
            def _():
                pltpu.async_copy(xtab.at[srcv.at[sp, j + 1]],
                                 rows2.at[npar], gsem.at[npar])

            pltpu.make_async_copy(xtab.at[srcv.at[sp, j]], rows2.at[par],
                                  gsem.at[par]).wait()

            for g in range(CHUNK // 16):
                pvec = pv[sp, j, pl.ds(g * 16, 16)]
                for i in range(16):
                    e = g * 16 + i
                    pb = jnp.full((16,), pvec[i], _F32)
                    for q in range(DH // 16):
                        scaled2[par, e, pl.ds(q * 16, 16)] = (
                            rows2[par, e, pl.ds(q * 16, 16)] * pb)

            pltpu.async_copy(rows2.at[par], acc_sum.at[dstv.at[sp, j]],
                             ssem.at[par], add=True)
            pltpu.async_copy(scaled2.at[par], acc_p.at[dstv.at[sp, j]],
                             psem.at[par], add=True)

            @pl.when(lax.rem(b * SB + j, 2) == c)
            def _():
                pltpu.async_copy(ones, acc_deg.at[dstv.at[sp, j]],
                                 dsem.at[par], add=True)

            return carry2

        lax.fori_loop(0, SB, chunk_body, 0)
        _wait_scatters(sp, (SB - 1) % 2, SB - 1, b)
        return carry

    lax.fori_loop(0, NSB, super_body, 0)

    plsc.subcore_barrier()

    r0 = s * ROWS_PT
    pltpu.sync_copy(acc_sum.at[pl.ds(r0, ROWS_PT)],
                    out_sum.at[c, pl.ds(r0, ROWS_PT)])
    pltpu.sync_copy(acc_p.at[pl.ds(r0, ROWS_PT)],
                    out_p.at[c, pl.ds(r0, ROWS_PT)])

    @pl.when(s == 0)
    def _():
        pltpu.sync_copy(acc_sum.at[pl.ds(NT * ROWS_PT, TAIL)],
                        out_sum.at[c, pl.ds(NT * ROWS_PT, TAIL)])
        pltpu.sync_copy(acc_p.at[pl.ds(NT * ROWS_PT, TAIL)],
                        out_p.at[c, pl.ds(NT * ROWS_PT, TAIL)])

    @pl.when(jnp.logical_and(c == 0, s < 7))
    def _():
        pltpu.sync_copy(acc_deg.at[pl.ds(s * 1280, 1280)],
                        out_deg_a.at[pl.ds(s * 1280, 1280)])

    @pl.when(jnp.logical_and(c == 0, s == 7))
    def _():
        pltpu.sync_copy(acc_deg.at[pl.ds(7 * 1280, 1040)],
                        out_deg_a.at[pl.ds(7 * 1280, 1040)])

    @pl.when(jnp.logical_and(c == 1, s < 7))
    def _():
        pltpu.sync_copy(acc_deg.at[pl.ds(s * 1280, 1280)],
                        out_deg_b.at[pl.ds(s * 1280, 1280)])

    @pl.when(jnp.logical_and(c == 1, s == 7))
    def _():
        pltpu.sync_copy(acc_deg.at[pl.ds(7 * 1280, 1040)],
                        out_deg_b.at[pl.ds(7 * 1280, 1040)])


def _sc_scatter(xtab, ei4, p2):
    mesh = plsc.VectorSubcoreMesh(core_axis_name="c", subcore_axis_name="s")
    f = pl.kernel(
        _sc_body,
        mesh=mesh,
        compiler_params=pltpu.CompilerParams(needs_layout_passes=False,
                                             use_tc_tiling_on_sc=False),
        out_type=[
            jax.ShapeDtypeStruct((NC, N_NODES, DH), _F32),
            jax.ShapeDtypeStruct((NC, N_NODES, DH), _F32),
            jax.ShapeDtypeStruct((N_NODES,), _F32),
            jax.ShapeDtypeStruct((N_NODES,), _F32),
        ],
        scratch_types=[
            pltpu.VMEM_SHARED((N_NODES + PAD_ROWS, DH), _F32),
            pltpu.VMEM_SHARED((N_NODES + PAD_ROWS, DH), _F32),
            pltpu.VMEM_SHARED((N_NODES + PAD_ROWS,), _F32),
            pltpu.VMEM((2, SB, CHUNK), _I32),
            pltpu.VMEM((2, SB, CHUNK), _I32),
            pltpu.VMEM((2, SB, CHUNK), _F32),
            pltpu.VMEM((2, CHUNK, DH), _F32),
            pltpu.VMEM((2, CHUNK, DH), _F32),
            pltpu.VMEM((ZR, DH), _F32),
            pltpu.VMEM((1280,), _F32),
            pltpu.VMEM((CHUNK,), _F32),
            pltpu.SemaphoreType.DMA((2,)),
            pltpu.SemaphoreType.DMA((2,)),
            pltpu.SemaphoreType.DMA((2,)),
            pltpu.SemaphoreType.DMA((2,)),
            pltpu.SemaphoreType.DMA((2,)),
        ],
    )
    return f(xtab, ei4, p2)


def _tc_body(s2, pa2, dga, dgb, xr, wcat, wroot, brow, out):
    cat = jnp.concatenate([s2[0], s2[1], pa2[0], pa2[1]], axis=1)
    deg = dga[...] + dgb[...]
    agg = jnp.dot(cat, wcat[...], preferred_element_type=_F32)
    agg = agg / jnp.maximum(deg, 1.0)
    out[...] = (agg
                + jnp.dot(xr[...], wroot[...], preferred_element_type=_F32)
                + brow[...])


def _tc_combine(sum2, p2, dega, degb, x, wcat, wroot, brow):
    grid = (N_NODES // RB,)
    return pl.pallas_call(
        _tc_body,
        grid=grid,
        in_specs=[
            pl.BlockSpec((NC, RB, DH), lambda i: (0, i, 0)),
            pl.BlockSpec((NC, RB, DH), lambda i: (0, i, 0)),
            pl.BlockSpec((RB, 1), lambda i: (i, 0)),
            pl.BlockSpec((RB, 1), lambda i: (i, 0)),
            pl.BlockSpec((RB, D), lambda i: (i, 0)),
            pl.BlockSpec((2 * D, D), lambda i: (0, 0)),
            pl.BlockSpec((D, D), lambda i: (0, 0)),
            pl.BlockSpec((1, D), lambda i: (0, 0)),
        ],
        out_specs=pl.BlockSpec((RB, D), lambda i: (i, 0)),
        out_shape=jax.ShapeDtypeStruct((N_NODES, D), _F32),
    )(sum2, p2, dega, degb, x, wcat, wroot, brow)


def kernel(x, edge_index, edge_attr, weight, root, bias):
    p = edge_attr[:, 0]
    xtab = jnp.concatenate([x[:, :DH], x[:, DH:]], axis=0)
    npad = E_PAD - N_EDGES
    pidx = jnp.arange(npad, dtype=jnp.int32)
    pad_ei = jnp.stack([pidx % N_NODES, N_NODES + (pidx % PAD_ROWS)])
    ei4 = jnp.concatenate([edge_index, pad_ei], axis=1).reshape(
        2, NT, NCHUNK, CHUNK)
    p2 = jnp.concatenate([p, jnp.zeros((npad,), _F32)]).reshape(
        NT, NCHUNK, CHUNK)

    sum2, pacc2, deg_a, deg_b = _sc_scatter(xtab, ei4, p2)

    wcat = jnp.concatenate([weight[0], weight[1] - weight[0]], axis=0)
    return _tc_combine(sum2, pacc2,
                       deg_a.reshape(N_NODES, 1), deg_b.reshape(N_NODES, 1),
                       x, wcat, root, bias.reshape(1, D))
